# baseline (speedup 1.0000x reference)
import functools

import jax
import jax.numpy as jnp
from jax import lax
from jax.experimental import pallas as pl
from jax.experimental.pallas import tpu as pltpu
from jax.experimental.pallas import tpu_sc as plsc

_info = plsc.get_sparse_core_info()
_NC, _NS = _info.num_cores, _info.num_subcores
_NW = _NC * _NS
_L = _info.num_lanes

_BB = 128
_NSLOT = 5
_KG = 3


@functools.partial(jax.jit, static_argnames=("batch", "hist", "dim"))
def _sc_embed(idx_t, table, *, batch, hist, dim):
    bpw = batch // _NW
    blocks_per_l = bpw // _BB
    n_blocks = hist * blocks_per_l
    assert n_blocks % _NSLOT == 0
    mesh = plsc.VectorSubcoreMesh(core_axis_name="c", subcore_axis_name="s")

    @functools.partial(
        pl.kernel,
        mesh=mesh,
        out_type=jax.ShapeDtypeStruct((hist, dim, batch), jnp.float32),
        scratch_types=[
            pltpu.VMEM((hist, bpw), jnp.int32),
            pltpu.VMEM((_NSLOT, _BB, dim), jnp.float32),
            pltpu.VMEM((_NSLOT, dim, _BB), jnp.float32),
            pltpu.SemaphoreType.DMA,
            pltpu.SemaphoreType.DMA((_NSLOT,)),
            pltpu.SemaphoreType.DMA((_NSLOT,)),
        ],
        compiler_params=pltpu.CompilerParams(use_tc_tiling_on_sc=False,
                                             needs_layout_passes=False),
    )
    def k(idx_hbm, tab_hbm, out_hbm, idx_v, rows_v, tr_v, isem, gsem, wsem):
        wid = lax.axis_index("s") * _NC + lax.axis_index("c")
        b0 = wid * bpw

        pltpu.async_copy(idx_hbm.at[:, pl.ds(b0, bpw)], idx_v, isem).wait()

        iota = lax.iota(jnp.int32, _L)

        def fire_gather(l, j, s):
            pltpu.async_copy(
                tab_hbm.at[idx_v.at[l].at[pl.ds(j * _BB, _BB)]],
                rows_v.at[s], gsem.at[s])

        def wait_gather(l, j, s):
            pltpu.make_async_copy(
                tab_hbm.at[idx_v.at[l].at[pl.ds(j * _BB, _BB)]],
                rows_v.at[s], gsem.at[s]).wait()

        def fire_write(l, j, s):
            pltpu.async_copy(tr_v.at[s],
                             out_hbm.at[l].at[:, pl.ds(b0 + j * _BB, _BB)],
                             wsem.at[s])

        def wait_write(l, j, s):
            pltpu.make_async_copy(tr_v.at[s],
                                  out_hbm.at[l].at[:, pl.ds(b0 + j * _BB, _BB)],
                                  wsem.at[s]).wait()

        def lj(i):
            return i // blocks_per_l, i % blocks_per_l

        for i in range(_KG):
            fire_gather(*lj(i), i)

        def group(g, _):
            for t in range(_NSLOT):
                i = g * _NSLOT + t
                sx = (t + _KG) % _NSLOT

                @pl.when(i + _KG < n_blocks)
                def _():
                    x = i + _KG
                    fire_gather(x // blocks_per_l, x % blocks_per_l, sx)

                wait_gather(i // blocks_per_l, i % blocks_per_l, t)

                @pl.when(i - _NSLOT >= 0)
                def _():
                    p = i - _NSLOT
                    wait_write(p // blocks_per_l, p % blocks_per_l, t)

                for d in range(dim):
                    cidx = jnp.full((_L,), d, jnp.int32)
                    for kk in range(_BB // _L):
                        v = plsc.load_gather(rows_v.at[t],
                                             [iota + kk * _L, cidx])
                        tr_v[t, d, pl.ds(kk * _L, _L)] = v

                fire_write(i // blocks_per_l, i % blocks_per_l, t)
            return 0

        lax.fori_loop(0, n_blocks // _NSLOT, group, 0)

        for t in range(_NSLOT):
            i = n_blocks - _NSLOT + t
            wait_write(i // blocks_per_l, i % blocks_per_l, t)

    return k(idx_t, table)


def kernel(input, weight):
    B, L = input.shape
    V, D = weight.shape
    assert B % (_NW * _BB) == 0
    idx_t = input.T
    out = _sc_embed(idx_t, weight, batch=B, hist=L, dim=D)
    return out.transpose(2, 0, 1)


# speedup vs baseline: 1.3490x; 1.3490x over previous
import functools

import jax
import jax.numpy as jnp
from jax import lax
from jax.experimental import pallas as pl
from jax.experimental.pallas import tpu as pltpu
from jax.experimental.pallas import tpu_sc as plsc

_info = plsc.get_sparse_core_info()
_NC, _NS = _info.num_cores, _info.num_subcores
_NW = _NC * _NS
_L = _info.num_lanes

_BB = 128
_NSLOT = 5
_KG = 3


@functools.partial(jax.jit, static_argnames=("batch", "hist", "dim"))
def _sc_embed(idx_t, table, *, batch, hist, dim):
    bpw = batch // _NW
    blocks_per_l = bpw // _BB
    n_blocks = hist * blocks_per_l
    assert n_blocks % _NSLOT == 0
    mesh = plsc.VectorSubcoreMesh(core_axis_name="c", subcore_axis_name="s")

    @functools.partial(
        pl.kernel,
        mesh=mesh,
        out_type=jax.ShapeDtypeStruct((hist, batch, dim), jnp.float32),
        scratch_types=[
            pltpu.VMEM((hist, bpw), jnp.int32),
            pltpu.VMEM((_NSLOT, _BB, dim), jnp.float32),
            pltpu.SemaphoreType.DMA,
            pltpu.SemaphoreType.DMA((_NSLOT,)),
            pltpu.SemaphoreType.DMA((_NSLOT,)),
        ],
        compiler_params=pltpu.CompilerParams(use_tc_tiling_on_sc=False,
                                             needs_layout_passes=False),
    )
    def k(idx_hbm, tab_hbm, out_hbm, idx_v, rows_v, isem, gsem, wsem):
        wid = lax.axis_index("s") * _NC + lax.axis_index("c")
        b0 = wid * bpw

        pltpu.async_copy(idx_hbm.at[:, pl.ds(b0, bpw)], idx_v, isem).wait()

        iota = lax.iota(jnp.int32, _L)

        def fire_gather(l, j, s):
            pltpu.async_copy(
                tab_hbm.at[idx_v.at[l].at[pl.ds(j * _BB, _BB)]],
                rows_v.at[s], gsem.at[s])

        def wait_gather(l, j, s):
            pltpu.make_async_copy(
                tab_hbm.at[idx_v.at[l].at[pl.ds(j * _BB, _BB)]],
                rows_v.at[s], gsem.at[s]).wait()

        def fire_write(l, j, s):
            pltpu.async_copy(rows_v.at[s],
                             out_hbm.at[l].at[pl.ds(b0 + j * _BB, _BB)],
                             wsem.at[s])

        def wait_write(l, j, s):
            pltpu.make_async_copy(rows_v.at[s],
                                  out_hbm.at[l].at[pl.ds(b0 + j * _BB, _BB)],
                                  wsem.at[s]).wait()

        def lj(i):
            return i // blocks_per_l, i % blocks_per_l

        for i in range(_KG):
            fire_gather(*lj(i), i)

        def group(g, _):
            for t in range(_NSLOT):
                i = g * _NSLOT + t
                sx = (t + _KG) % _NSLOT

                @pl.when(jnp.logical_and(i + _KG < n_blocks,
                                         i + _KG - _NSLOT >= 0))
                def _():
                    p = i + _KG - _NSLOT
                    wait_write(p // blocks_per_l, p % blocks_per_l, sx)

                @pl.when(i + _KG < n_blocks)
                def _():
                    x = i + _KG
                    fire_gather(x // blocks_per_l, x % blocks_per_l, sx)

                wait_gather(i // blocks_per_l, i % blocks_per_l, t)

                fire_write(i // blocks_per_l, i % blocks_per_l, t)
            return 0

        lax.fori_loop(0, n_blocks // _NSLOT, group, 0)

        for t in range(_NSLOT):
            i = n_blocks - _NSLOT + t
            wait_write(i // blocks_per_l, i % blocks_per_l, t)

    return k(idx_t, table)


def kernel(input, weight):
    B, L = input.shape
    V, D = weight.shape
    assert B % (_NW * _BB) == 0
    idx_t = input.T
    out = _sc_embed(idx_t, weight, batch=B, hist=L, dim=D)
    return out.transpose(1, 0, 2)
